# baseline (speedup 1.0000x reference)
import functools

import jax
import jax.numpy as jnp
from jax import lax
from jax.experimental import pallas as pl
from jax.experimental.pallas import tpu as pltpu
from jax.experimental.pallas import tpu_sc as plsc

_PAIR_I = (0, 0, 0, 1, 1, 2)
_PAIR_J = (1, 2, 3, 2, 3, 3)
_NP = len(_PAIR_I)
_N = 8192
_ROWS = 1024
_CHUNKS = 4
_CSC = 1024
_NTC = _N - _CSC
_NSUB = 32
_QPW = _N // _NSUB
_QV = 16
_QBLK = 4


def _tc_body(pa_ref, pb_ref, a_ref, b_ref, out_ref, baug_ref):
    p = pl.program_id(0)
    r = pl.program_id(1)

    @pl.when(r == 0)
    def _build_b():
        bx = b_ref[0, 0:1, :]
        by = b_ref[0, 1:2, :]
        bz = b_ref[0, 2:3, :]
        wbx = pb_ref[p, 0] * bx + pb_ref[p, 1] * by + pb_ref[p, 2] * bz + pb_ref[p, 3]
        wby = pb_ref[p, 4] * bx + pb_ref[p, 5] * by + pb_ref[p, 6] * bz + pb_ref[p, 7]
        wbz = pb_ref[p, 8] * bx + pb_ref[p, 9] * by + pb_ref[p, 10] * bz + pb_ref[p, 11]
        baug_ref[0:1, :] = -2.0 * wbx
        baug_ref[1:2, :] = -2.0 * wby
        baug_ref[2:3, :] = -2.0 * wbz
        baug_ref[3:4, :] = wbx * wbx + wby * wby + wbz * wbz

    ax = a_ref[0, 0:1, :]
    ay = a_ref[0, 1:2, :]
    az = a_ref[0, 2:3, :]
    wax = pa_ref[p, 0] * ax + pa_ref[p, 1] * ay + pa_ref[p, 2] * az + pa_ref[p, 3]
    way = pa_ref[p, 4] * ax + pa_ref[p, 5] * ay + pa_ref[p, 6] * az + pa_ref[p, 7]
    waz = pa_ref[p, 8] * ax + pa_ref[p, 9] * ay + pa_ref[p, 10] * az + pa_ref[p, 11]
    a2 = wax * wax + way * way + waz * waz
    aaug = jnp.concatenate([wax, way, waz, jnp.ones_like(wax)], axis=0)

    chunk = _NTC // _CHUNKS
    m = None
    for k in range(_CHUNKS):
        ht = jax.lax.dot_general(
            baug_ref[:, k * chunk:(k + 1) * chunk], aaug,
            dimension_numbers=(((0,), (0,)), ((), ())),
            preferred_element_type=jnp.float32,
        )
        mk = jnp.min(ht, axis=0, keepdims=True)
        m = mk if m is None else jnp.minimum(m, mk)

    out_ref[pl.ds(p, 1), pl.ds(r * _ROWS, _ROWS)] = m + a2


def _sc_body(a_hbm, b_hbm, pa_hbm, pb_hbm, out_hbm,
             pav, pbv, ar0, ar1, ar2, br0, br1, br2,
             bgx, bgy, bgz, bg2, waxv, wayv, wazv, wa2v, tout):
    wid = lax.axis_index("s") * 2 + lax.axis_index("c")
    base = wid * _QPW
    pltpu.sync_copy(pa_hbm, pav)
    pltpu.sync_copy(pb_hbm, pbv)

    def _pair(p, carry):
        pltpu.sync_copy(b_hbm.at[pl.ds((p * 3 + 0) * _CSC, _CSC)], br0)
        pltpu.sync_copy(b_hbm.at[pl.ds((p * 3 + 1) * _CSC, _CSC)], br1)
        pltpu.sync_copy(b_hbm.at[pl.ds((p * 3 + 2) * _CSC, _CSC)], br2)
        pltpu.sync_copy(a_hbm.at[pl.ds((p * 3 + 0) * _N + base, _QPW)], ar0)
        pltpu.sync_copy(a_hbm.at[pl.ds((p * 3 + 1) * _N + base, _QPW)], ar1)
        pltpu.sync_copy(a_hbm.at[pl.ds((p * 3 + 2) * _N + base, _QPW)], ar2)
        pa_vec = pav[pl.ds(p * 16, 16)]
        pb_vec = pbv[pl.ds(p * 16, 16)]

        def _bbuild(i, c2):
            s = pl.ds(i * _QV, _QV)
            bx = br0[s]
            by = br1[s]
            bz = br2[s]
            wbx = pb_vec[0] * bx + pb_vec[1] * by + pb_vec[2] * bz + pb_vec[3]
            wby = pb_vec[4] * bx + pb_vec[5] * by + pb_vec[6] * bz + pb_vec[7]
            wbz = pb_vec[8] * bx + pb_vec[9] * by + pb_vec[10] * bz + pb_vec[11]
            bgx[s] = -2.0 * wbx
            bgy[s] = -2.0 * wby
            bgz[s] = -2.0 * wbz
            bg2[s] = wbx * wbx + wby * wby + wbz * wbz
            return c2

        lax.fori_loop(0, _CSC // _QV, _bbuild, 0)

        def _qbuild(i, c2):
            s = pl.ds(i * _QV, _QV)
            ax = ar0[s]
            ay = ar1[s]
            az = ar2[s]
            wax = pa_vec[0] * ax + pa_vec[1] * ay + pa_vec[2] * az + pa_vec[3]
            way = pa_vec[4] * ax + pa_vec[5] * ay + pa_vec[6] * az + pa_vec[7]
            waz = pa_vec[8] * ax + pa_vec[9] * ay + pa_vec[10] * az + pa_vec[11]
            waxv[s] = wax
            wayv[s] = way
            wazv[s] = waz
            wa2v[s] = wax * wax + way * way + waz * waz
            return c2

        lax.fori_loop(0, _QPW // _QV, _qbuild, 0)

        big = jnp.full((_QV,), 3.0e38, dtype=jnp.float32)

        def _qblock(qb, c2):
            qbase = qb * (_QV * _QBLK)
            qs = [pl.ds(qbase + j * _QV, _QV) for j in range(_QBLK)]
            qx = [waxv[s] for s in qs]
            qy = [wayv[s] for s in qs]
            qz = [wazv[s] for s in qs]

            def _cols(cv, ms):
                s = pl.ds(cv * _QV, _QV)
                bxv = bgx[s]
                byv = bgy[s]
                bzv = bgz[s]
                b2v = bg2[s]
                for lane in range(_QV):
                    bxs = bxv[lane]
                    bys = byv[lane]
                    bzs = bzv[lane]
                    b2s = b2v[lane]
                    ms = tuple(
                        jnp.minimum(ms[j], (b2s + qx[j] * bxs) + (qy[j] * bys + qz[j] * bzs))
                        for j in range(_QBLK)
                    )
                return ms

            ms = lax.fori_loop(0, _CSC // _QV, _cols, (big,) * _QBLK)
            for j in range(_QBLK):
                tout[qs[j]] = ms[j] + wa2v[qs[j]]
            return c2

        lax.fori_loop(0, _QPW // (_QV * _QBLK), _qblock, 0)
        pltpu.sync_copy(tout, out_hbm.at[pl.ds(p * _N + base, _QPW)])
        return carry

    lax.fori_loop(0, _NP, _pair, 0)


def _combine_body(ttc_ref, tsc_ref, out_ref):
    m = jnp.minimum(ttc_ref[...], tsc_ref[...])
    d = jnp.sqrt(jnp.maximum(m, 1e-12))
    out_ref[0, 0] = jnp.sum(d) * (1.0 / (6.0 * _N))


def kernel(point_clouds, camera_poses):
    idx_i = jnp.array(_PAIR_I)
    idx_j = jnp.array(_PAIR_J)
    pc_t = jnp.transpose(point_clouds, (0, 2, 1))
    a_in = pc_t[idx_i]
    b_in = pc_t[idx_j]
    b_tc = b_in[:, :, :_NTC]
    b_sc = b_in[:, :, _NTC:]
    pose_rows = camera_poses[:, :3, :].reshape(4, 12)
    pa = pose_rows[idx_i]
    pb = pose_rows[idx_j]
    pa_flat = jnp.pad(pa, ((0, 0), (0, 4))).reshape(_NP * 16)
    pb_flat = jnp.pad(pb, ((0, 0), (0, 4))).reshape(_NP * 16)

    nr = _N // _ROWS
    t_tc = pl.pallas_call(
        _tc_body,
        grid=(_NP, nr),
        in_specs=[
            pl.BlockSpec(memory_space=pltpu.SMEM),
            pl.BlockSpec(memory_space=pltpu.SMEM),
            pl.BlockSpec((1, 3, _ROWS), lambda p, r: (p, 0, r)),
            pl.BlockSpec((1, 3, _NTC), lambda p, r: (p, 0, 0)),
        ],
        out_specs=pl.BlockSpec((_NP, _N), lambda p, r: (0, 0)),
        out_shape=jax.ShapeDtypeStruct((_NP, _N), jnp.float32),
        scratch_shapes=[pltpu.VMEM((4, _NTC), jnp.float32)],
    )(pa, pb, a_in, b_tc)

    sc_kernel = functools.partial(
        pl.kernel,
        mesh=plsc.VectorSubcoreMesh(core_axis_name="c", subcore_axis_name="s"),
        out_type=jax.ShapeDtypeStruct((_NP * _N,), jnp.float32),
        scratch_types=[
            pltpu.VMEM((_NP * 16,), jnp.float32),
            pltpu.VMEM((_NP * 16,), jnp.float32),
            pltpu.VMEM((_QPW,), jnp.float32),
            pltpu.VMEM((_QPW,), jnp.float32),
            pltpu.VMEM((_QPW,), jnp.float32),
            pltpu.VMEM((_CSC,), jnp.float32),
            pltpu.VMEM((_CSC,), jnp.float32),
            pltpu.VMEM((_CSC,), jnp.float32),
            pltpu.VMEM((_CSC,), jnp.float32),
            pltpu.VMEM((_CSC,), jnp.float32),
            pltpu.VMEM((_CSC,), jnp.float32),
            pltpu.VMEM((_CSC,), jnp.float32),
            pltpu.VMEM((_QPW,), jnp.float32),
            pltpu.VMEM((_QPW,), jnp.float32),
            pltpu.VMEM((_QPW,), jnp.float32),
            pltpu.VMEM((_QPW,), jnp.float32),
            pltpu.VMEM((_QPW,), jnp.float32),
        ],
    )(_sc_body)
    t_sc = sc_kernel(a_in.reshape(-1), b_sc.reshape(-1), pa_flat, pb_flat)

    out = pl.pallas_call(
        _combine_body,
        out_specs=pl.BlockSpec(memory_space=pltpu.SMEM),
        out_shape=jax.ShapeDtypeStruct((1, 1), jnp.float32),
    )(t_tc, t_sc.reshape(_NP, _N))
    return out[0, 0]


# speedup vs baseline: 4.7305x; 4.7305x over previous
import functools

import jax
import jax.numpy as jnp
from jax import lax
from jax.experimental import pallas as pl
from jax.experimental.pallas import tpu as pltpu
from jax.experimental.pallas import tpu_sc as plsc

_PAIR_I = (0, 0, 0, 1, 1, 2)
_PAIR_J = (1, 2, 3, 2, 3, 3)
_NP = len(_PAIR_I)
_N = 8192
_ROWS = 1024
_CHUNKS = 4
_CSC = 1024
_NTC = _N - _CSC
_NSUB = 32
_QPW = _N // _NSUB
_QV = 16
_QBLK = 8


def _tc_body(pa_ref, pb_ref, a_ref, b_ref, out_ref, baug_ref):
    p = pl.program_id(0)
    r = pl.program_id(1)

    @pl.when(r == 0)
    def _build_b():
        bx = b_ref[0, 0:1, :]
        by = b_ref[0, 1:2, :]
        bz = b_ref[0, 2:3, :]
        wbx = pb_ref[p, 0] * bx + pb_ref[p, 1] * by + pb_ref[p, 2] * bz + pb_ref[p, 3]
        wby = pb_ref[p, 4] * bx + pb_ref[p, 5] * by + pb_ref[p, 6] * bz + pb_ref[p, 7]
        wbz = pb_ref[p, 8] * bx + pb_ref[p, 9] * by + pb_ref[p, 10] * bz + pb_ref[p, 11]
        baug_ref[0:1, :] = -2.0 * wbx
        baug_ref[1:2, :] = -2.0 * wby
        baug_ref[2:3, :] = -2.0 * wbz
        baug_ref[3:4, :] = wbx * wbx + wby * wby + wbz * wbz

    ax = a_ref[0, 0:1, :]
    ay = a_ref[0, 1:2, :]
    az = a_ref[0, 2:3, :]
    wax = pa_ref[p, 0] * ax + pa_ref[p, 1] * ay + pa_ref[p, 2] * az + pa_ref[p, 3]
    way = pa_ref[p, 4] * ax + pa_ref[p, 5] * ay + pa_ref[p, 6] * az + pa_ref[p, 7]
    waz = pa_ref[p, 8] * ax + pa_ref[p, 9] * ay + pa_ref[p, 10] * az + pa_ref[p, 11]
    a2 = wax * wax + way * way + waz * waz
    aaug = jnp.concatenate([wax, way, waz, jnp.ones_like(wax)], axis=0)

    chunk = _NTC // _CHUNKS
    m = None
    for k in range(_CHUNKS):
        ht = jax.lax.dot_general(
            baug_ref[:, k * chunk:(k + 1) * chunk], aaug,
            dimension_numbers=(((0,), (0,)), ((), ())),
            preferred_element_type=jnp.float32,
        )
        mk = jnp.min(ht, axis=0, keepdims=True)
        m = mk if m is None else jnp.minimum(m, mk)

    out_ref[pl.ds(p, 1), pl.ds(r * _ROWS, _ROWS)] = m + a2


def _sc_body(a_hbm, b_hbm, pa_hbm, pb_hbm, out_hbm,
             pav, pbv, ar0, ar1, ar2, br0, br1, br2,
             bgx, bgy, bgz, bg2, waxv, wayv, wazv, wa2v, tout):
    wid = lax.axis_index("s") * 2 + lax.axis_index("c")
    base = wid * _QPW
    pltpu.sync_copy(pa_hbm, pav)
    pltpu.sync_copy(pb_hbm, pbv)

    def _pair(p, carry):
        pltpu.sync_copy(b_hbm.at[pl.ds((p * 3 + 0) * _CSC, _CSC)], br0)
        pltpu.sync_copy(b_hbm.at[pl.ds((p * 3 + 1) * _CSC, _CSC)], br1)
        pltpu.sync_copy(b_hbm.at[pl.ds((p * 3 + 2) * _CSC, _CSC)], br2)
        pltpu.sync_copy(a_hbm.at[pl.ds((p * 3 + 0) * _N + base, _QPW)], ar0)
        pltpu.sync_copy(a_hbm.at[pl.ds((p * 3 + 1) * _N + base, _QPW)], ar1)
        pltpu.sync_copy(a_hbm.at[pl.ds((p * 3 + 2) * _N + base, _QPW)], ar2)
        pa_vec = pav[pl.ds(p * 16, 16)]
        pb_vec = pbv[pl.ds(p * 16, 16)]

        def _bbuild(i, c2):
            s = pl.ds(i * _QV, _QV)
            bx = br0[s]
            by = br1[s]
            bz = br2[s]
            wbx = pb_vec[0] * bx + pb_vec[1] * by + pb_vec[2] * bz + pb_vec[3]
            wby = pb_vec[4] * bx + pb_vec[5] * by + pb_vec[6] * bz + pb_vec[7]
            wbz = pb_vec[8] * bx + pb_vec[9] * by + pb_vec[10] * bz + pb_vec[11]
            bgx[s] = -2.0 * wbx
            bgy[s] = -2.0 * wby
            bgz[s] = -2.0 * wbz
            bg2[s] = wbx * wbx + wby * wby + wbz * wbz
            return c2

        lax.fori_loop(0, _CSC // _QV, _bbuild, 0)

        def _qbuild(i, c2):
            s = pl.ds(i * _QV, _QV)
            ax = ar0[s]
            ay = ar1[s]
            az = ar2[s]
            wax = pa_vec[0] * ax + pa_vec[1] * ay + pa_vec[2] * az + pa_vec[3]
            way = pa_vec[4] * ax + pa_vec[5] * ay + pa_vec[6] * az + pa_vec[7]
            waz = pa_vec[8] * ax + pa_vec[9] * ay + pa_vec[10] * az + pa_vec[11]
            waxv[s] = wax
            wayv[s] = way
            wazv[s] = waz
            wa2v[s] = wax * wax + way * way + waz * waz
            return c2

        lax.fori_loop(0, _QPW // _QV, _qbuild, 0)

        big = jnp.full((_QV,), 3.0e38, dtype=jnp.float32)
        lane_iota = jnp.arange(_QV, dtype=jnp.int32)

        def _qvloop(qv, c2):
            s = pl.ds(qv * _QV, _QV)
            qxv = waxv[s]
            qyv = wayv[s]
            qzv = wazv[s]
            qa2 = wa2v[s]
            res = jnp.zeros((_QV,), dtype=jnp.float32)
            for half in range(_QV // _QBLK):
                lanes = [half * _QBLK + j for j in range(_QBLK)]
                xs = [qxv[l] for l in lanes]
                ys = [qyv[l] for l in lanes]
                zs = [qzv[l] for l in lanes]

                def _cols(cv, mv):
                    cs = pl.ds(cv * _QV, _QV)
                    bxv = bgx[cs]
                    byv = bgy[cs]
                    bzv = bgz[cs]
                    b2v = bg2[cs]
                    return tuple(
                        jnp.minimum(mv[j], (b2v + xs[j] * bxv) + (ys[j] * byv + zs[j] * bzv))
                        for j in range(_QBLK)
                    )

                mv = lax.fori_loop(0, _CSC // _QV, _cols, (big,) * _QBLK)
                for j in range(_QBLK):
                    m = mv[j]
                    for sh in (8, 4, 2, 1):
                        idx = lane_iota ^ sh
                        m = jnp.minimum(m, m.at[idx].get(mode="promise_in_bounds"))
                    res = jnp.where(lane_iota == lanes[j], m + qa2[lanes[j]], res)
            tout[s] = res
            return c2

        lax.fori_loop(0, _QPW // _QV, _qvloop, 0)
        pltpu.sync_copy(tout, out_hbm.at[pl.ds(p * _N + base, _QPW)])
        return carry

    lax.fori_loop(0, _NP, _pair, 0)


def _combine_body(ttc_ref, tsc_ref, out_ref):
    m = jnp.minimum(ttc_ref[...], tsc_ref[...])
    d = jnp.sqrt(jnp.maximum(m, 1e-12))
    out_ref[0, 0] = jnp.sum(d) * (1.0 / (6.0 * _N))


def kernel(point_clouds, camera_poses):
    idx_i = jnp.array(_PAIR_I)
    idx_j = jnp.array(_PAIR_J)
    pc_t = jnp.transpose(point_clouds, (0, 2, 1))
    a_in = pc_t[idx_i]
    b_in = pc_t[idx_j]
    b_tc = b_in[:, :, :_NTC]
    b_sc = b_in[:, :, _NTC:]
    pose_rows = camera_poses[:, :3, :].reshape(4, 12)
    pa = pose_rows[idx_i]
    pb = pose_rows[idx_j]
    pa_flat = jnp.pad(pa, ((0, 0), (0, 4))).reshape(_NP * 16)
    pb_flat = jnp.pad(pb, ((0, 0), (0, 4))).reshape(_NP * 16)

    nr = _N // _ROWS
    t_tc = pl.pallas_call(
        _tc_body,
        grid=(_NP, nr),
        in_specs=[
            pl.BlockSpec(memory_space=pltpu.SMEM),
            pl.BlockSpec(memory_space=pltpu.SMEM),
            pl.BlockSpec((1, 3, _ROWS), lambda p, r: (p, 0, r)),
            pl.BlockSpec((1, 3, _NTC), lambda p, r: (p, 0, 0)),
        ],
        out_specs=pl.BlockSpec((_NP, _N), lambda p, r: (0, 0)),
        out_shape=jax.ShapeDtypeStruct((_NP, _N), jnp.float32),
        scratch_shapes=[pltpu.VMEM((4, _NTC), jnp.float32)],
    )(pa, pb, a_in, b_tc)

    sc_kernel = functools.partial(
        pl.kernel,
        mesh=plsc.VectorSubcoreMesh(core_axis_name="c", subcore_axis_name="s"),
        out_type=jax.ShapeDtypeStruct((_NP * _N,), jnp.float32),
        scratch_types=[
            pltpu.VMEM((_NP * 16,), jnp.float32),
            pltpu.VMEM((_NP * 16,), jnp.float32),
            pltpu.VMEM((_QPW,), jnp.float32),
            pltpu.VMEM((_QPW,), jnp.float32),
            pltpu.VMEM((_QPW,), jnp.float32),
            pltpu.VMEM((_CSC,), jnp.float32),
            pltpu.VMEM((_CSC,), jnp.float32),
            pltpu.VMEM((_CSC,), jnp.float32),
            pltpu.VMEM((_CSC,), jnp.float32),
            pltpu.VMEM((_CSC,), jnp.float32),
            pltpu.VMEM((_CSC,), jnp.float32),
            pltpu.VMEM((_CSC,), jnp.float32),
            pltpu.VMEM((_QPW,), jnp.float32),
            pltpu.VMEM((_QPW,), jnp.float32),
            pltpu.VMEM((_QPW,), jnp.float32),
            pltpu.VMEM((_QPW,), jnp.float32),
            pltpu.VMEM((_QPW,), jnp.float32),
        ],
    )(_sc_body)
    t_sc = sc_kernel(a_in.reshape(-1), b_sc.reshape(-1), pa_flat, pb_flat)

    out = pl.pallas_call(
        _combine_body,
        out_specs=pl.BlockSpec(memory_space=pltpu.SMEM),
        out_shape=jax.ShapeDtypeStruct((1, 1), jnp.float32),
    )(t_tc, t_sc.reshape(_NP, _N))
    return out[0, 0]
